# Initial kernel scaffold; baseline (speedup 1.0000x reference)
import jax
import jax.numpy as jnp
from jax.experimental import pallas as pl


def kernel(feat_0, feat_1, feat_2, feat_3, feat_4, feat_5, feat_6, feat_7, feat_8, feat_9, feat_10, feat_11, feat_12, feat_13, feat_14, feat_15, feat_16, feat_17, feat_18, feat_19, feat_20, feat_21, feat_22, feat_23, feat_24, feat_25, emb_tables, wide_weights):
    raise NotImplementedError("write your pallas kernel here")


# speedup vs baseline: 1.2961x; 1.2961x over previous
import functools

import jax
import jax.numpy as jnp
from jax import lax
from jax.experimental import pallas as pl
from jax.experimental.pallas import tpu as pltpu
from jax.experimental.pallas import tpu_sc as plsc

_B = 16384
_F = 26
_E = 32
_BUCKET = 100000
_R = _B * _F
_NC = 2
_NS = 16
_NW = _NC * _NS
_L = 16
_GROUP = 128
_GPW = _R // _GROUP // _NW
_G_PER_CHUNK = 8
_CHUNK = _G_PER_CHUNK * _GROUP
_NCHUNK = _GPW // _G_PER_CHUNK
_WROWS = _F * _BUCKET // _L


@functools.cache
def _sc_gather_fn():
    mesh = plsc.VectorSubcoreMesh(
        core_axis_name="c", subcore_axis_name="s", num_cores=_NC,
        num_subcores=_NS)

    @functools.partial(
        pl.kernel,
        out_type=(
            jax.ShapeDtypeStruct((_R, _E), jnp.float32),
            jax.ShapeDtypeStruct((_R,), jnp.float32),
        ),
        mesh=mesh,
        scratch_types=[
            pltpu.VMEM((_GPW, _GROUP), jnp.int32),
            pltpu.VMEM((_GPW, _GROUP), jnp.int32),
            pltpu.VMEM((_CHUNK, _E), jnp.float32),
            pltpu.VMEM((_CHUNK, _L), jnp.float32),
            pltpu.VMEM((_CHUNK,), jnp.float32),
            pltpu.SemaphoreType.DMA,
            pltpu.SemaphoreType.DMA,
        ],
        compiler_params=pltpu.CompilerParams(
            use_tc_tiling_on_sc=False, needs_layout_passes=False),
    )
    def sc_gather(gidx_hbm, widx_hbm, table_hbm, wide_hbm, emb_out, wide_out,
                  idx_v, widx_v, rows_v, wrows_v, wvals_v, sem_e, sem_w):
        wid = lax.axis_index("s") * _NC + lax.axis_index("c")
        gbase = wid * _GPW
        pltpu.sync_copy(gidx_hbm.at[pl.ds(gbase, _GPW)], idx_v)
        pltpu.sync_copy(widx_hbm.at[pl.ds(gbase, _GPW)], widx_v)
        lane = lax.iota(jnp.int32, _L)

        def chunk_body(c, carry):
            copies = []
            for g in range(_G_PER_CHUNK):
                row = c * _G_PER_CHUNK + g
                copies.append(pltpu.async_copy(
                    table_hbm.at[idx_v.at[row]],
                    rows_v.at[pl.ds(g * _GROUP, _GROUP)], sem_e))
                copies.append(pltpu.async_copy(
                    wide_hbm.at[widx_v.at[row]],
                    wrows_v.at[pl.ds(g * _GROUP, _GROUP)], sem_w))
            for cp in copies:
                cp.wait()
            for g in range(_G_PER_CHUNK):
                row = c * _G_PER_CHUNK + g
                for j in range(_GROUP // _L):
                    cols = idx_v[row, pl.ds(j * _L, _L)] & (_L - 1)
                    sel = plsc.load_gather(
                        wrows_v, [g * _GROUP + j * _L + lane, cols])
                    wvals_v[pl.ds(g * _GROUP + j * _L, _L)] = sel
            out_base = (gbase + c * _G_PER_CHUNK) * _GROUP
            pltpu.sync_copy(rows_v, emb_out.at[pl.ds(out_base, _CHUNK)])
            pltpu.sync_copy(wvals_v, wide_out.at[pl.ds(out_base, _CHUNK)])
            return carry

        lax.fori_loop(0, _NCHUNK, chunk_body, 0)

    return sc_gather


def kernel(feat_0, feat_1, feat_2, feat_3, feat_4, feat_5, feat_6, feat_7,
           feat_8, feat_9, feat_10, feat_11, feat_12, feat_13, feat_14,
           feat_15, feat_16, feat_17, feat_18, feat_19, feat_20, feat_21,
           feat_22, feat_23, feat_24, feat_25, emb_tables, wide_weights):
    feats = jnp.concatenate(
        [feat_0, feat_1, feat_2, feat_3, feat_4, feat_5, feat_6, feat_7,
         feat_8, feat_9, feat_10, feat_11, feat_12, feat_13, feat_14,
         feat_15, feat_16, feat_17, feat_18, feat_19, feat_20, feat_21,
         feat_22, feat_23, feat_24, feat_25], axis=1)
    offs = (jnp.arange(_F, dtype=jnp.int32) * _BUCKET)[None, :]
    gidx = (feats + offs).reshape(_R // _GROUP, _GROUP)
    widx = gidx >> 4
    table_flat = emb_tables.reshape(_F * _BUCKET, _E)
    wide_flat = wide_weights.reshape(_WROWS, _L)
    emb_flat, wide_vals = _sc_gather_fn()(gidx, widx, table_flat, wide_flat)
    return (wide_vals.reshape(_B, _F), emb_flat.reshape(_B, _F, _E))


# speedup vs baseline: 4.7618x; 3.6739x over previous
import functools

import jax
import jax.numpy as jnp
from jax import lax
from jax.experimental import pallas as pl
from jax.experimental.pallas import tpu as pltpu
from jax.experimental.pallas import tpu_sc as plsc

_B = 16384
_F = 26
_E = 32
_BUCKET = 100000
_NC = 2
_NS = 16
_NW = _NC * _NS
_L = 16
_H = _B // 2


@functools.cache
def _sc_gather_fn():
    mesh = plsc.VectorSubcoreMesh(
        core_axis_name="c", subcore_axis_name="s", num_cores=_NC,
        num_subcores=_NS)

    @functools.partial(
        pl.kernel,
        out_type=(
            jax.ShapeDtypeStruct((_F * _E, _B), jnp.float32),
            jax.ShapeDtypeStruct((_F, _B), jnp.float32),
        ),
        mesh=mesh,
        scratch_types=[
            pltpu.VMEM((_BUCKET,), jnp.float32),
            pltpu.VMEM((_H,), jnp.int32),
            pltpu.VMEM((_H,), jnp.float32),
        ],
        compiler_params=pltpu.CompilerParams(
            use_tc_tiling_on_sc=True, needs_layout_passes=False),
    )
    def sc_gather(feats_hbm, tt_hbm, wt_hbm, emb_out, wide_out,
                  rowbuf, idxbuf, outbuf):
        e = lax.axis_index("c") * _NS + lax.axis_index("s")

        def gather_half(k, _):
            ids = idxbuf[pl.ds(k * _L, _L)]
            outbuf[pl.ds(k * _L, _L)] = plsc.load_gather(rowbuf, [ids])
            return _

        for f in range(_F):
            row = f * _E + e
            pltpu.sync_copy(tt_hbm.at[row], rowbuf)
            for h in range(2):
                pltpu.sync_copy(feats_hbm.at[f, pl.ds(h * _H, _H)], idxbuf)
                lax.fori_loop(0, _H // _L, gather_half, 0)
                pltpu.sync_copy(outbuf, emb_out.at[row, pl.ds(h * _H, _H)])

        @pl.when(e < _F)
        def _wide():
            pltpu.sync_copy(wt_hbm.at[e], rowbuf)
            for h in range(2):
                pltpu.sync_copy(feats_hbm.at[e, pl.ds(h * _H, _H)], idxbuf)
                lax.fori_loop(0, _H // _L, gather_half, 0)
                pltpu.sync_copy(outbuf, wide_out.at[e, pl.ds(h * _H, _H)])

    return sc_gather


def kernel(feat_0, feat_1, feat_2, feat_3, feat_4, feat_5, feat_6, feat_7,
           feat_8, feat_9, feat_10, feat_11, feat_12, feat_13, feat_14,
           feat_15, feat_16, feat_17, feat_18, feat_19, feat_20, feat_21,
           feat_22, feat_23, feat_24, feat_25, emb_tables, wide_weights):
    feats = jnp.stack(
        [feat_0[:, 0], feat_1[:, 0], feat_2[:, 0], feat_3[:, 0],
         feat_4[:, 0], feat_5[:, 0], feat_6[:, 0], feat_7[:, 0],
         feat_8[:, 0], feat_9[:, 0], feat_10[:, 0], feat_11[:, 0],
         feat_12[:, 0], feat_13[:, 0], feat_14[:, 0], feat_15[:, 0],
         feat_16[:, 0], feat_17[:, 0], feat_18[:, 0], feat_19[:, 0],
         feat_20[:, 0], feat_21[:, 0], feat_22[:, 0], feat_23[:, 0],
         feat_24[:, 0], feat_25[:, 0]], axis=0)
    tt = emb_tables.transpose(0, 2, 1).reshape(_F * _E, _BUCKET)
    emb_t, wide_t = _sc_gather_fn()(feats, tt, wide_weights)
    emb = emb_t.reshape(_F, _E, _B).transpose(2, 0, 1)
    wide = wide_t.transpose(1, 0)
    return (wide, emb)


# speedup vs baseline: 5.1601x; 1.0836x over previous
import functools

import jax
import jax.numpy as jnp
from jax import lax
from jax.experimental import pallas as pl
from jax.experimental.pallas import tpu as pltpu
from jax.experimental.pallas import tpu_sc as plsc

_B = 16384
_F = 26
_E = 32
_BUCKET = 100000
_NC = 2
_NS = 16
_NW = _NC * _NS
_L = 16
_Q = 4096
_NQ = _B // _Q


@functools.cache
def _sc_gather_fn():
    mesh = plsc.VectorSubcoreMesh(
        core_axis_name="c", subcore_axis_name="s", num_cores=_NC,
        num_subcores=_NS)

    @functools.partial(
        pl.kernel,
        out_type=(
            jax.ShapeDtypeStruct((_F * _E, _B), jnp.float32),
            jax.ShapeDtypeStruct((_F, _B), jnp.float32),
        ),
        mesh=mesh,
        scratch_types=[
            pltpu.VMEM((_BUCKET,), jnp.float32),
            pltpu.VMEM((2, _Q), jnp.int32),
            pltpu.VMEM((_B,), jnp.float32),
            pltpu.SemaphoreType.DMA,
            pltpu.SemaphoreType.DMA,
        ],
        compiler_params=pltpu.CompilerParams(
            use_tc_tiling_on_sc=True, needs_layout_passes=False),
    )
    def sc_gather(feats_hbm, tt_hbm, wt_hbm, emb_out, wide_out,
                  rowbuf, idxq, part, sem_i, sem_o):
        e = lax.axis_index("c") * _NS + lax.axis_index("s")

        def idx_copy(f, q, s):
            return pltpu.make_async_copy(
                feats_hbm.at[f, pl.ds(q * _Q, _Q)], idxq.at[s], sem_i)

        def out_copy(row, q):
            return pltpu.make_async_copy(
                part.at[pl.ds(q * _Q, _Q)],
                emb_out.at[row, pl.ds(q * _Q, _Q)], sem_o)

        def gather_quarter(s, qbase):
            def kbody(k, _):
                base = qbase + k * 2 * _L
                ids0 = idxq[s, pl.ds(k * 2 * _L, _L)]
                ids1 = idxq[s, pl.ds(k * 2 * _L + _L, _L)]
                part[pl.ds(base, _L)] = plsc.load_gather(rowbuf, [ids0])
                part[pl.ds(base + _L, _L)] = plsc.load_gather(rowbuf, [ids1])
                return _
            lax.fori_loop(0, _Q // (2 * _L), kbody, 0)

        def field_body(f, carry):
            row = f * _E + e
            pltpu.sync_copy(tt_hbm.at[row], rowbuf)

            @pl.when(f > 0)
            def _wait_prev_out():
                for q in range(_NQ):
                    out_copy(row, q).wait()

            for q in range(_NQ):
                idx_copy(f, q, q % 2).wait()
                if q < _NQ - 1:
                    idx_copy(f, q + 1, (q + 1) % 2).start()
                gather_quarter(q % 2, q * _Q)
                out_copy(row, q).start()

            @pl.when(f < _F - 1)
            def _prefetch_idx():
                idx_copy(f + 1, 0, 0).start()

            return carry

        idx_copy(0, 0, 0).start()
        lax.fori_loop(0, _F, field_body, 0)
        for q in range(_NQ):
            out_copy((_F - 1) * _E + e, q).wait()

        @pl.when(e < _F)
        def _wide():
            pltpu.sync_copy(wt_hbm.at[e], rowbuf)
            for q in range(_NQ):
                pltpu.sync_copy(feats_hbm.at[e, pl.ds(q * _Q, _Q)],
                                idxq.at[0])
                gather_quarter(0, q * _Q)
                pltpu.sync_copy(part.at[pl.ds(q * _Q, _Q)],
                                wide_out.at[e, pl.ds(q * _Q, _Q)])

    return sc_gather


def kernel(feat_0, feat_1, feat_2, feat_3, feat_4, feat_5, feat_6, feat_7,
           feat_8, feat_9, feat_10, feat_11, feat_12, feat_13, feat_14,
           feat_15, feat_16, feat_17, feat_18, feat_19, feat_20, feat_21,
           feat_22, feat_23, feat_24, feat_25, emb_tables, wide_weights):
    feats = jnp.stack(
        [feat_0[:, 0], feat_1[:, 0], feat_2[:, 0], feat_3[:, 0],
         feat_4[:, 0], feat_5[:, 0], feat_6[:, 0], feat_7[:, 0],
         feat_8[:, 0], feat_9[:, 0], feat_10[:, 0], feat_11[:, 0],
         feat_12[:, 0], feat_13[:, 0], feat_14[:, 0], feat_15[:, 0],
         feat_16[:, 0], feat_17[:, 0], feat_18[:, 0], feat_19[:, 0],
         feat_20[:, 0], feat_21[:, 0], feat_22[:, 0], feat_23[:, 0],
         feat_24[:, 0], feat_25[:, 0]], axis=0)
    tt = emb_tables.transpose(0, 2, 1).reshape(_F * _E, _BUCKET)
    emb_t, wide_t = _sc_gather_fn()(feats, tt, wide_weights)
    emb = emb_t.reshape(_F, _E, _B).transpose(2, 0, 1)
    wide = wide_t.transpose(1, 0)
    return (wide, emb)


# speedup vs baseline: 7.4643x; 1.4466x over previous
import functools

import jax
import jax.numpy as jnp
from jax import lax
from jax.experimental import pallas as pl
from jax.experimental.pallas import tpu as pltpu
from jax.experimental.pallas import tpu_sc as plsc

_B = 16384
_F = 26
_E = 32
_BUCKET = 100000
_NC = 2
_NS = 16
_NW = _NC * _NS
_L = 16
_Q = 4096
_NQ = _B // _Q


@functools.cache
def _sc_gather_fn():
    mesh = plsc.VectorSubcoreMesh(
        core_axis_name="c", subcore_axis_name="s", num_cores=_NC,
        num_subcores=_NS)

    @functools.partial(
        pl.kernel,
        out_type=(
            jax.ShapeDtypeStruct((_F * _E, _B), jnp.float32),
            jax.ShapeDtypeStruct((_F, _B), jnp.float32),
        ),
        mesh=mesh,
        scratch_types=[
            pltpu.VMEM((_BUCKET,), jnp.float32),
            pltpu.VMEM((2, _Q), jnp.int32),
            pltpu.VMEM((_B,), jnp.float32),
            pltpu.SemaphoreType.DMA,
            pltpu.SemaphoreType.DMA,
        ],
        compiler_params=pltpu.CompilerParams(
            use_tc_tiling_on_sc=True, needs_layout_passes=False),
    )
    def sc_gather(feats_hbm, tt_hbm, wt_hbm, emb_out, wide_out,
                  rowbuf, idxq, part, sem_i, sem_o):
        e = lax.axis_index("c") * _NS + lax.axis_index("s")

        def idx_copy(f, q, s):
            return pltpu.make_async_copy(
                feats_hbm.at[f, pl.ds(q * _Q, _Q)], idxq.at[s], sem_i)

        def out_copy(row, q):
            return pltpu.make_async_copy(
                part.at[pl.ds(q * _Q, _Q)],
                emb_out.at[row, pl.ds(q * _Q, _Q)], sem_o)

        def gather_quarter(s, qbase):
            @plsc.parallel_loop(0, _Q // _L, unroll=4)
            def _loop(k):
                ids = idxq[s, pl.ds(k * _L, _L)]
                part[pl.ds(qbase + k * _L, _L)] = plsc.load_gather(
                    rowbuf, [ids])

        def field_body(f, carry):
            row = f * _E + e
            pltpu.sync_copy(tt_hbm.at[row], rowbuf)

            @pl.when(f > 0)
            def _wait_prev_out():
                for q in range(_NQ):
                    out_copy(row, q).wait()

            for q in range(_NQ):
                idx_copy(f, q, q % 2).wait()
                if q < _NQ - 1:
                    idx_copy(f, q + 1, (q + 1) % 2).start()
                gather_quarter(q % 2, q * _Q)
                out_copy(row, q).start()

            @pl.when(f < _F - 1)
            def _prefetch_idx():
                idx_copy(f + 1, 0, 0).start()

            return carry

        idx_copy(0, 0, 0).start()
        lax.fori_loop(0, _F, field_body, 0)
        for q in range(_NQ):
            out_copy((_F - 1) * _E + e, q).wait()

        @pl.when(e < _F)
        def _wide():
            pltpu.sync_copy(wt_hbm.at[e], rowbuf)
            for q in range(_NQ):
                pltpu.sync_copy(feats_hbm.at[e, pl.ds(q * _Q, _Q)],
                                idxq.at[0])
                gather_quarter(0, q * _Q)
                pltpu.sync_copy(part.at[pl.ds(q * _Q, _Q)],
                                wide_out.at[e, pl.ds(q * _Q, _Q)])

    return sc_gather


def kernel(feat_0, feat_1, feat_2, feat_3, feat_4, feat_5, feat_6, feat_7,
           feat_8, feat_9, feat_10, feat_11, feat_12, feat_13, feat_14,
           feat_15, feat_16, feat_17, feat_18, feat_19, feat_20, feat_21,
           feat_22, feat_23, feat_24, feat_25, emb_tables, wide_weights):
    feats = jnp.stack(
        [feat_0[:, 0], feat_1[:, 0], feat_2[:, 0], feat_3[:, 0],
         feat_4[:, 0], feat_5[:, 0], feat_6[:, 0], feat_7[:, 0],
         feat_8[:, 0], feat_9[:, 0], feat_10[:, 0], feat_11[:, 0],
         feat_12[:, 0], feat_13[:, 0], feat_14[:, 0], feat_15[:, 0],
         feat_16[:, 0], feat_17[:, 0], feat_18[:, 0], feat_19[:, 0],
         feat_20[:, 0], feat_21[:, 0], feat_22[:, 0], feat_23[:, 0],
         feat_24[:, 0], feat_25[:, 0]], axis=0)
    tt = emb_tables.transpose(0, 2, 1).reshape(_F * _E, _BUCKET)
    emb_t, wide_t = _sc_gather_fn()(feats, tt, wide_weights)
    emb = emb_t.reshape(_F, _E, _B).transpose(2, 0, 1)
    wide = wide_t.transpose(1, 0)
    return (wide, emb)


# speedup vs baseline: 7.4843x; 1.0027x over previous
import functools

import jax
import jax.numpy as jnp
from jax import lax
from jax.experimental import pallas as pl
from jax.experimental.pallas import tpu as pltpu
from jax.experimental.pallas import tpu_sc as plsc

_B = 16384
_F = 26
_E = 32
_BUCKET = 100000
_NC = 2
_NS = 16
_NW = _NC * _NS
_L = 16
_Q = 4096
_NQ = _B // _Q


@functools.cache
def _sc_gather_fn():
    mesh = plsc.VectorSubcoreMesh(
        core_axis_name="c", subcore_axis_name="s", num_cores=_NC,
        num_subcores=_NS)

    @functools.partial(
        pl.kernel,
        out_type=(
            jax.ShapeDtypeStruct((_F * _E, _B), jnp.float32),
            jax.ShapeDtypeStruct((_F, _B), jnp.float32),
        ),
        mesh=mesh,
        scratch_types=[
            pltpu.VMEM((_BUCKET,), jnp.float32),
            pltpu.VMEM((2, _Q), jnp.int32),
            pltpu.VMEM((_B,), jnp.float32),
            pltpu.SemaphoreType.DMA,
            pltpu.SemaphoreType.DMA,
        ],
        compiler_params=pltpu.CompilerParams(
            use_tc_tiling_on_sc=True, needs_layout_passes=False),
    )
    def sc_gather(feats_hbm, tt_hbm, wt_hbm, emb_out, wide_out,
                  rowbuf, idxq, part, sem_i, sem_o):
        e = lax.axis_index("c") * _NS + lax.axis_index("s")

        def idx_copy(f, q, s):
            return pltpu.make_async_copy(
                feats_hbm.at[f, pl.ds(q * _Q, _Q)], idxq.at[s], sem_i)

        def out_copy(row, q):
            return pltpu.make_async_copy(
                part.at[pl.ds(q * _Q, _Q)],
                emb_out.at[row, pl.ds(q * _Q, _Q)], sem_o)

        def gather_quarter(s, qbase):
            @plsc.parallel_loop(0, _Q // _L, unroll=8)
            def _loop(k):
                ids = idxq[s, pl.ds(k * _L, _L)]
                part[pl.ds(qbase + k * _L, _L)] = plsc.load_gather(
                    rowbuf, [ids])

        def field_body(f, carry):
            row = f * _E + e
            pltpu.sync_copy(tt_hbm.at[row], rowbuf)

            @pl.when(f > 0)
            def _wait_prev_out():
                for q in range(_NQ):
                    out_copy(row, q).wait()

            for q in range(_NQ):
                idx_copy(f, q, q % 2).wait()
                if q < _NQ - 1:
                    idx_copy(f, q + 1, (q + 1) % 2).start()
                gather_quarter(q % 2, q * _Q)
                out_copy(row, q).start()

            @pl.when(f < _F - 1)
            def _prefetch_idx():
                idx_copy(f + 1, 0, 0).start()

            return carry

        idx_copy(0, 0, 0).start()
        lax.fori_loop(0, _F, field_body, 0)
        for q in range(_NQ):
            out_copy((_F - 1) * _E + e, q).wait()

        @pl.when(e < _F)
        def _wide():
            pltpu.sync_copy(wt_hbm.at[e], rowbuf)
            for q in range(_NQ):
                pltpu.sync_copy(feats_hbm.at[e, pl.ds(q * _Q, _Q)],
                                idxq.at[0])
                gather_quarter(0, q * _Q)
                pltpu.sync_copy(part.at[pl.ds(q * _Q, _Q)],
                                wide_out.at[e, pl.ds(q * _Q, _Q)])

    return sc_gather


def kernel(feat_0, feat_1, feat_2, feat_3, feat_4, feat_5, feat_6, feat_7,
           feat_8, feat_9, feat_10, feat_11, feat_12, feat_13, feat_14,
           feat_15, feat_16, feat_17, feat_18, feat_19, feat_20, feat_21,
           feat_22, feat_23, feat_24, feat_25, emb_tables, wide_weights):
    feats = jnp.stack(
        [feat_0[:, 0], feat_1[:, 0], feat_2[:, 0], feat_3[:, 0],
         feat_4[:, 0], feat_5[:, 0], feat_6[:, 0], feat_7[:, 0],
         feat_8[:, 0], feat_9[:, 0], feat_10[:, 0], feat_11[:, 0],
         feat_12[:, 0], feat_13[:, 0], feat_14[:, 0], feat_15[:, 0],
         feat_16[:, 0], feat_17[:, 0], feat_18[:, 0], feat_19[:, 0],
         feat_20[:, 0], feat_21[:, 0], feat_22[:, 0], feat_23[:, 0],
         feat_24[:, 0], feat_25[:, 0]], axis=0)
    tt = emb_tables.transpose(0, 2, 1).reshape(_F * _E, _BUCKET)
    emb_t, wide_t = _sc_gather_fn()(feats, tt, wide_weights)
    emb = emb_t.reshape(_F, _E, _B).transpose(2, 0, 1)
    wide = wide_t.transpose(1, 0)
    return (wide, emb)
